# Initial kernel scaffold; baseline (speedup 1.0000x reference)
import jax
import jax.numpy as jnp
from jax.experimental import pallas as pl


def kernel(f0, f1, f2, f3, f4, f5, f6, f7, f8, f9, f10, f11, f12, f13, f14, f15, f16, f17, f18, f19, f20, f21, f22, f23, f24, f25, W0, W1, W2, W3, W4, W5, W6, W7, W8, W9, W10, W11, W12, W13, W14, W15, W16, W17, W18, W19, W20, W21, W22, W23, W24, W25):
    raise NotImplementedError("write your pallas kernel here")


# speedup vs baseline: 3.7583x; 3.7583x over previous
import functools

import jax
import jax.numpy as jnp
from jax import lax
from jax.experimental import pallas as pl
from jax.experimental.pallas import tpu as pltpu
from jax.experimental.pallas import tpu_sc as plsc

_NUM_FIELDS = 26
_VOCAB = 100000
_DIM = 128
_BATCH = 4096
_NUM_WORKERS = 32
_BPW = _BATCH // _NUM_WORKERS


def _build_kernel():
    mesh = plsc.VectorSubcoreMesh(core_axis_name="c", subcore_axis_name="s")

    @functools.partial(
        pl.kernel,
        mesh=mesh,
        out_type=jax.ShapeDtypeStruct((_BATCH, _NUM_FIELDS * _DIM), jnp.float32),
        scratch_types=[
            pltpu.VMEM((_NUM_FIELDS, _BPW), jnp.int32),
            pltpu.VMEM((_BPW, _DIM), jnp.float32),
            pltpu.VMEM((_BPW, _DIM), jnp.float32),
            pltpu.SemaphoreType.DMA,
            pltpu.SemaphoreType.DMA,
            pltpu.SemaphoreType.DMA,
            pltpu.SemaphoreType.DMA,
        ],
    )
    def k(idx_hbm, *rest):
        ws = rest[:_NUM_FIELDS]
        out_hbm = rest[_NUM_FIELDS]
        idx_v, buf0, buf1, g0, g1, w0, w1 = rest[_NUM_FIELDS + 1:]
        bufs = (buf0, buf1)
        gsems = (g0, g1)
        wsems = (w0, w1)

        wid = lax.axis_index("s") * 2 + lax.axis_index("c")
        base = wid * _BPW
        pltpu.sync_copy(idx_hbm.at[wid], idx_v)

        gathers = []
        for f in range(_NUM_FIELDS):
            gathers.append(
                pltpu.make_async_copy(
                    ws[f].at[idx_v.at[f]], bufs[f % 2], gsems[f % 2]
                )
            )
        writes = []
        for f in range(_NUM_FIELDS):
            writes.append(
                pltpu.make_async_copy(
                    bufs[f % 2],
                    out_hbm.at[pl.ds(base, _BPW), pl.ds(f * _DIM, _DIM)],
                    wsems[f % 2],
                )
            )

        gathers[0].start()
        for f in range(_NUM_FIELDS):
            if f + 1 < _NUM_FIELDS:
                if f >= 1:
                    writes[f - 1].wait()
                gathers[f + 1].start()
            gathers[f].wait()
            writes[f].start()
        writes[_NUM_FIELDS - 2].wait()
        writes[_NUM_FIELDS - 1].wait()

    return k


_kernel_call = _build_kernel()


@jax.jit
def kernel(f0, f1, f2, f3, f4, f5, f6, f7, f8, f9, f10, f11, f12, f13, f14,
           f15, f16, f17, f18, f19, f20, f21, f22, f23, f24, f25,
           W0, W1, W2, W3, W4, W5, W6, W7, W8, W9, W10, W11, W12, W13, W14,
           W15, W16, W17, W18, W19, W20, W21, W22, W23, W24, W25):
    idxs = [f0, f1, f2, f3, f4, f5, f6, f7, f8, f9, f10, f11, f12, f13, f14,
            f15, f16, f17, f18, f19, f20, f21, f22, f23, f24, f25]
    ws = [W0, W1, W2, W3, W4, W5, W6, W7, W8, W9, W10, W11, W12, W13, W14,
          W15, W16, W17, W18, W19, W20, W21, W22, W23, W24, W25]
    idx = jnp.stack(idxs, axis=0).astype(jnp.int32)
    idx = idx.reshape(_NUM_FIELDS, _NUM_WORKERS, _BPW).transpose(1, 0, 2)
    return _kernel_call(idx, *ws)


# speedup vs baseline: 3.8272x; 1.0183x over previous
import functools

import jax
import jax.numpy as jnp
from jax import lax
from jax.experimental import pallas as pl
from jax.experimental.pallas import tpu as pltpu
from jax.experimental.pallas import tpu_sc as plsc

_NUM_FIELDS = 26
_VOCAB = 100000
_DIM = 128
_BATCH = 4096
_NUM_WORKERS = 32
_BPW = _BATCH // _NUM_WORKERS


def _build_kernel():
    mesh = plsc.VectorSubcoreMesh(core_axis_name="c", subcore_axis_name="s")

    @functools.partial(
        pl.kernel,
        mesh=mesh,
        out_type=jax.ShapeDtypeStruct((_BATCH, _NUM_FIELDS * _DIM), jnp.float32),
        scratch_types=[
            pltpu.VMEM((_NUM_FIELDS, _BPW), jnp.int32),
            pltpu.VMEM((_BPW, _DIM), jnp.float32),
            pltpu.VMEM((_BPW, _DIM), jnp.float32),
            pltpu.SemaphoreType.DMA,
            pltpu.SemaphoreType.DMA,
            pltpu.SemaphoreType.DMA,
            pltpu.SemaphoreType.DMA,
            pltpu.SemaphoreType.DMA,
        ],
    )
    def k(*rest):
        idxs_hbm = rest[:_NUM_FIELDS]
        ws = rest[_NUM_FIELDS:2 * _NUM_FIELDS]
        out_hbm = rest[2 * _NUM_FIELDS]
        idx_v, buf0, buf1, g0, g1, w0, w1, isem = rest[2 * _NUM_FIELDS + 1:]
        bufs = (buf0, buf1)
        gsems = (g0, g1)
        wsems = (w0, w1)

        wid = lax.axis_index("s") * 2 + lax.axis_index("c")
        base = wid * _BPW
        icopies = [
            pltpu.make_async_copy(
                idxs_hbm[f].at[pl.ds(base, _BPW)], idx_v.at[f], isem
            )
            for f in range(_NUM_FIELDS)
        ]
        for c in icopies:
            c.start()
        for c in icopies:
            c.wait()

        gathers = []
        for f in range(_NUM_FIELDS):
            gathers.append(
                pltpu.make_async_copy(
                    ws[f].at[idx_v.at[f]], bufs[f % 2], gsems[f % 2]
                )
            )
        writes = []
        for f in range(_NUM_FIELDS):
            writes.append(
                pltpu.make_async_copy(
                    bufs[f % 2],
                    out_hbm.at[pl.ds(base, _BPW), pl.ds(f * _DIM, _DIM)],
                    wsems[f % 2],
                )
            )

        gathers[0].start()
        for f in range(_NUM_FIELDS):
            if f + 1 < _NUM_FIELDS:
                if f >= 1:
                    writes[f - 1].wait()
                gathers[f + 1].start()
            gathers[f].wait()
            writes[f].start()
        writes[_NUM_FIELDS - 2].wait()
        writes[_NUM_FIELDS - 1].wait()

    return k


_kernel_call = _build_kernel()


@jax.jit
def kernel(f0, f1, f2, f3, f4, f5, f6, f7, f8, f9, f10, f11, f12, f13, f14,
           f15, f16, f17, f18, f19, f20, f21, f22, f23, f24, f25,
           W0, W1, W2, W3, W4, W5, W6, W7, W8, W9, W10, W11, W12, W13, W14,
           W15, W16, W17, W18, W19, W20, W21, W22, W23, W24, W25):
    idxs = [f0, f1, f2, f3, f4, f5, f6, f7, f8, f9, f10, f11, f12, f13, f14,
            f15, f16, f17, f18, f19, f20, f21, f22, f23, f24, f25]
    ws = [W0, W1, W2, W3, W4, W5, W6, W7, W8, W9, W10, W11, W12, W13, W14,
          W15, W16, W17, W18, W19, W20, W21, W22, W23, W24, W25]
    idxs = [i.astype(jnp.int32) for i in idxs]
    return _kernel_call(*idxs, *ws)


# speedup vs baseline: 3.9329x; 1.0276x over previous
import functools

import jax
import jax.numpy as jnp
from jax import lax
from jax.experimental import pallas as pl
from jax.experimental.pallas import tpu as pltpu
from jax.experimental.pallas import tpu_sc as plsc

_NUM_FIELDS = 26
_VOCAB = 100000
_DIM = 128
_BATCH = 4096
_NUM_WORKERS = 32
_BPW = _BATCH // _NUM_WORKERS


def _build_kernel():
    mesh = plsc.VectorSubcoreMesh(core_axis_name="c", subcore_axis_name="s")

    @functools.partial(
        pl.kernel,
        mesh=mesh,
        out_type=jax.ShapeDtypeStruct((_BATCH, _NUM_FIELDS * _DIM), jnp.float32),
        scratch_types=[
            pltpu.VMEM((_NUM_FIELDS, _BPW), jnp.int32),
            pltpu.VMEM((_BPW, _DIM), jnp.float32),
            pltpu.VMEM((_BPW, _DIM), jnp.float32),
            pltpu.VMEM((_BPW, _DIM), jnp.float32),
            pltpu.VMEM((_BPW, _DIM), jnp.float32),
            pltpu.SemaphoreType.DMA,
            pltpu.SemaphoreType.DMA,
            pltpu.SemaphoreType.DMA,
            pltpu.SemaphoreType.DMA,
            pltpu.SemaphoreType.DMA,
            pltpu.SemaphoreType.DMA,
            pltpu.SemaphoreType.DMA,
            pltpu.SemaphoreType.DMA,
            pltpu.SemaphoreType.DMA,
        ],
    )
    def k(*rest):
        idxs_hbm = rest[:_NUM_FIELDS]
        ws = rest[_NUM_FIELDS:2 * _NUM_FIELDS]
        out_hbm = rest[2 * _NUM_FIELDS]
        (idx_v, buf0, buf1, buf2, buf3, g0, g1, g2, g3,
         w0, w1, w2, w3, isem) = rest[2 * _NUM_FIELDS + 1:]
        bufs = (buf0, buf1, buf2, buf3)
        gsems = (g0, g1, g2, g3)
        wsems = (w0, w1, w2, w3)

        wid = lax.axis_index("s") * 2 + lax.axis_index("c")
        base = wid * _BPW
        icopies = [
            pltpu.make_async_copy(
                idxs_hbm[f].at[pl.ds(base, _BPW)], idx_v.at[f], isem
            )
            for f in range(_NUM_FIELDS)
        ]
        for c in icopies:
            c.start()
        for c in icopies:
            c.wait()

        nbuf = 4
        gathers = []
        for f in range(_NUM_FIELDS):
            gathers.append(
                pltpu.make_async_copy(
                    ws[f].at[idx_v.at[f]], bufs[f % nbuf], gsems[f % nbuf]
                )
            )
        writes = []
        for f in range(_NUM_FIELDS):
            writes.append(
                pltpu.make_async_copy(
                    bufs[f % nbuf],
                    out_hbm.at[pl.ds(base, _BPW), pl.ds(f * _DIM, _DIM)],
                    wsems[f % nbuf],
                )
            )

        for f in range(min(nbuf, _NUM_FIELDS)):
            gathers[f].start()
        for f in range(_NUM_FIELDS):
            gathers[f].wait()
            writes[f].start()
            if f + nbuf < _NUM_FIELDS:
                writes[f].wait()
                gathers[f + nbuf].start()
        for f in range(max(0, _NUM_FIELDS - nbuf), _NUM_FIELDS):
            writes[f].wait()

    return k


_kernel_call = _build_kernel()


@jax.jit
def kernel(f0, f1, f2, f3, f4, f5, f6, f7, f8, f9, f10, f11, f12, f13, f14,
           f15, f16, f17, f18, f19, f20, f21, f22, f23, f24, f25,
           W0, W1, W2, W3, W4, W5, W6, W7, W8, W9, W10, W11, W12, W13, W14,
           W15, W16, W17, W18, W19, W20, W21, W22, W23, W24, W25):
    idxs = [f0, f1, f2, f3, f4, f5, f6, f7, f8, f9, f10, f11, f12, f13, f14,
            f15, f16, f17, f18, f19, f20, f21, f22, f23, f24, f25]
    ws = [W0, W1, W2, W3, W4, W5, W6, W7, W8, W9, W10, W11, W12, W13, W14,
          W15, W16, W17, W18, W19, W20, W21, W22, W23, W24, W25]
    idxs = [i.astype(jnp.int32) for i in idxs]
    return _kernel_call(*idxs, *ws)


# speedup vs baseline: 4.0002x; 1.0171x over previous
import functools

import jax
import jax.numpy as jnp
from jax import lax
from jax.experimental import pallas as pl
from jax.experimental.pallas import tpu as pltpu
from jax.experimental.pallas import tpu_sc as plsc

_NUM_FIELDS = 26
_VOCAB = 100000
_DIM = 128
_BATCH = 4096
_NUM_WORKERS = 32
_BPW = _BATCH // _NUM_WORKERS
_NBUF = 7


def _build_kernel():
    mesh = plsc.VectorSubcoreMesh(core_axis_name="c", subcore_axis_name="s")

    @functools.partial(
        pl.kernel,
        mesh=mesh,
        out_type=jax.ShapeDtypeStruct((_BATCH, _NUM_FIELDS * _DIM), jnp.float32),
        scratch_types=(
            [pltpu.VMEM((_NUM_FIELDS, _BPW), jnp.int32)]
            + [pltpu.VMEM((_BPW, _DIM), jnp.float32)] * _NBUF
            + [pltpu.SemaphoreType.DMA] * (2 * _NBUF + 1)
        ),
    )
    def k(*rest):
        idxs_hbm = rest[:_NUM_FIELDS]
        ws = rest[_NUM_FIELDS:2 * _NUM_FIELDS]
        out_hbm = rest[2 * _NUM_FIELDS]
        scratch = rest[2 * _NUM_FIELDS + 1:]
        idx_v = scratch[0]
        bufs = scratch[1:1 + _NBUF]
        gsems = scratch[1 + _NBUF:1 + 2 * _NBUF]
        wsems = scratch[1 + 2 * _NBUF:1 + 3 * _NBUF]
        isem = scratch[1 + 3 * _NBUF]

        wid = lax.axis_index("s") * 2 + lax.axis_index("c")
        base = wid * _BPW
        icopies = [
            pltpu.make_async_copy(
                idxs_hbm[f].at[pl.ds(base, _BPW)], idx_v.at[f], isem
            )
            for f in range(_NUM_FIELDS)
        ]
        for c in icopies:
            c.start()
        for c in icopies:
            c.wait()

        nbuf = _NBUF
        gathers = []
        for f in range(_NUM_FIELDS):
            gathers.append(
                pltpu.make_async_copy(
                    ws[f].at[idx_v.at[f]], bufs[f % nbuf], gsems[f % nbuf]
                )
            )
        writes = []
        for f in range(_NUM_FIELDS):
            writes.append(
                pltpu.make_async_copy(
                    bufs[f % nbuf],
                    out_hbm.at[pl.ds(base, _BPW), pl.ds(f * _DIM, _DIM)],
                    wsems[f % nbuf],
                )
            )

        for f in range(min(nbuf, _NUM_FIELDS)):
            gathers[f].start()
        for f in range(_NUM_FIELDS):
            gathers[f].wait()
            writes[f].start()
            if f + nbuf < _NUM_FIELDS:
                writes[f].wait()
                gathers[f + nbuf].start()
        for f in range(max(0, _NUM_FIELDS - nbuf), _NUM_FIELDS):
            writes[f].wait()

    return k


_kernel_call = _build_kernel()


@jax.jit
def kernel(f0, f1, f2, f3, f4, f5, f6, f7, f8, f9, f10, f11, f12, f13, f14,
           f15, f16, f17, f18, f19, f20, f21, f22, f23, f24, f25,
           W0, W1, W2, W3, W4, W5, W6, W7, W8, W9, W10, W11, W12, W13, W14,
           W15, W16, W17, W18, W19, W20, W21, W22, W23, W24, W25):
    idxs = [f0, f1, f2, f3, f4, f5, f6, f7, f8, f9, f10, f11, f12, f13, f14,
            f15, f16, f17, f18, f19, f20, f21, f22, f23, f24, f25]
    ws = [W0, W1, W2, W3, W4, W5, W6, W7, W8, W9, W10, W11, W12, W13, W14,
          W15, W16, W17, W18, W19, W20, W21, W22, W23, W24, W25]
    idxs = [i.astype(jnp.int32) for i in idxs]
    return _kernel_call(*idxs, *ws)
